# baseline (speedup 1.0000x reference)
import dataclasses
import functools

import jax
import jax.numpy as jnp
from jax.experimental import pallas as pl
from jax.experimental.pallas import tpu as pltpu
from jax.experimental.pallas import tpu_sc as plsc

NC = 2
NS = 16
LANES = 16
NW = NC * NS
B = 64
NSLOT = 3


def _sc_aggregate(x, row, col, chunks, n_pad):
    n, d = x.shape
    WB = 80
    nwb = n // WB
    wb_per = (nwb + NS - 1) // NS
    epw = chunks * B

    mesh = plsc.VectorSubcoreMesh(
        core_axis_name="c", subcore_axis_name="s",
        num_cores=NC, num_subcores=NS)

    cp = pltpu.CompilerParams()
    if "needs_layout_passes" in pltpu.CompilerParams.__dataclass_fields__:
        cp = dataclasses.replace(cp, needs_layout_passes=False)

    z_feat = jnp.zeros((WB, d), jnp.float32)

    @functools.partial(
        pl.kernel,
        out_type=(jax.ShapeDtypeStruct((NC * n, d), jnp.float32),
                  jax.ShapeDtypeStruct((NW, n_pad), jnp.float32)),
        mesh=mesh,
        compiler_params=cp,
        scratch_types=[
            [pltpu.VMEM((B,), jnp.int32)] * NSLOT,
            [pltpu.VMEM((B,), jnp.int32)] * NSLOT,
            [pltpu.VMEM((B, d), jnp.float32)] * NSLOT,
            pltpu.VMEM((n_pad,), jnp.float32),
            pltpu.VMEM_SHARED((n_pad, d), jnp.float32),
            [pltpu.SemaphoreType.DMA] * NSLOT,
            [pltpu.SemaphoreType.DMA] * NSLOT,
        ],
    )
    def agg_kernel(x_hbm, row_hbm, col_hbm, zf_hbm,
                   sum_hbm, deg_hbm, rb, cb, gb, dloc, acc,
                   sem_i, sem_g):
        cid = jax.lax.axis_index("c")
        sid = jax.lax.axis_index("s")
        wid = cid * NS + sid

        @pl.loop(0, wb_per)
        def _(k):
            blk = sid + k * NS

            @pl.when(blk < nwb)
            def _():
                pltpu.sync_copy(zf_hbm, acc.at[pl.ds(blk * WB, WB)])

        @pl.loop(0, n_pad, step=LANES)
        def _(j):
            dloc[pl.ds(j, LANES)] = jnp.zeros((LANES,), jnp.float32)

        plsc.subcore_barrier()

        base = wid * epw
        ones_v = jnp.ones((LANES,), jnp.float32)

        def idx_start(c, s):
            off = base + c * B
            pltpu.async_copy(row_hbm.at[pl.ds(off, B)], rb[s], sem_i[s])
            pltpu.async_copy(col_hbm.at[pl.ds(off, B)], cb[s], sem_i[s])

        def idx_wait(s):
            pltpu.make_async_copy(row_hbm.at[pl.ds(base, B)], rb[s],
                                  sem_i[s]).wait()
            pltpu.make_async_copy(col_hbm.at[pl.ds(base, B)], cb[s],
                                  sem_i[s]).wait()

        def gather_start(s):
            pltpu.async_copy(x_hbm.at[rb[s]], gb[s], sem_g[s])

        def gather_wait(s):
            pltpu.make_async_copy(x_hbm.at[rb[s]], gb[s], sem_g[s]).wait()

        def deg_update(s):
            @pl.loop(0, B, step=LANES)
            def _(j):
                plsc.addupdate_scatter(dloc, [cb[s][pl.ds(j, LANES)]], ones_v)

        for s in range(NSLOT):
            idx_start(s, s)
        for s in range(NSLOT):
            idx_wait(s)
            gather_start(s)

        def turn(c, s):
            gather_wait(s)
            pltpu.sync_copy(gb[s], acc.at[cb[s]], add=True)
            deg_update(s)

            @pl.when(c + NSLOT < chunks)
            def _():
                idx_start(c + NSLOT, s)
                idx_wait(s)
                gather_start(s)

        @pl.loop(0, chunks, step=NSLOT)
        def _(k):
            for s in range(NSLOT):
                turn(k + s, s)

        plsc.subcore_barrier()

        @pl.loop(0, wb_per)
        def _(k):
            blk = sid + k * NS

            @pl.when(blk < nwb)
            def _():
                pltpu.sync_copy(acc.at[pl.ds(blk * WB, WB)],
                                sum_hbm.at[pl.ds(cid * n + blk * WB, WB)])

        pltpu.sync_copy(dloc, deg_hbm.at[wid])

    return agg_kernel(x, row, col, z_feat)


def _tc_degsum(pdeg):
    n_pad = pdeg.shape[1]
    ones_nw = jnp.ones((NW, 1), jnp.float32)

    def body(dg_ref, on_ref, o_ref):
        deg = jax.lax.dot_general(
            dg_ref[...], on_ref[...], (((0,), (0,)), ((), ())),
            preferred_element_type=jnp.float32)
        o_ref[...] = jnp.maximum(deg, 1.0)

    return pl.pallas_call(
        body,
        out_shape=jax.ShapeDtypeStruct((n_pad, 1), jnp.float32),
    )(pdeg, ones_nw)


def _tc_combine(x, psum, deg, wt, b2):
    n, d = x.shape
    dout = wt.shape[1]
    bm = 1000
    grid = n // bm

    def body(x_ref, p0_ref, p1_ref, dg_ref, wt_ref, b_ref, o_ref):
        aggr = (p0_ref[...] + p1_ref[...]) / dg_ref[...]
        cat = jnp.concatenate([x_ref[...], aggr], axis=1)
        o_ref[...] = jnp.dot(cat, wt_ref[...],
                             preferred_element_type=jnp.float32) + b_ref[...]

    return pl.pallas_call(
        body,
        grid=(grid,),
        in_specs=[
            pl.BlockSpec((bm, d), lambda i: (i, 0)),
            pl.BlockSpec((bm, d), lambda i: (i, 0)),
            pl.BlockSpec((bm, d), lambda i, g=grid: (i + g, 0)),
            pl.BlockSpec((bm, 1), lambda i: (i, 0)),
            pl.BlockSpec((2 * d, dout), lambda i: (0, 0)),
            pl.BlockSpec((1, dout), lambda i: (0, 0)),
        ],
        out_specs=pl.BlockSpec((bm, dout), lambda i: (i, 0)),
        out_shape=jax.ShapeDtypeStruct((n, dout), jnp.float32),
    )(x, psum, psum, deg, wt, b2)


def kernel(x, edge_index, W, b):
    n = x.shape[0]
    e = edge_index.shape[1]
    row = edge_index[0].astype(jnp.int32)
    col = edge_index[1].astype(jnp.int32)

    unit = NW * B * NSLOT
    e_pad = ((e + unit - 1) // unit) * unit
    n_pad = n + LANES
    if e_pad != e:
        pad = e_pad - e
        row = jnp.concatenate([row, jnp.zeros((pad,), jnp.int32)])
        col = jnp.concatenate([col, jnp.full((pad,), n, jnp.int32)])
    chunks = e_pad // (NW * B)

    psum, pdeg = _sc_aggregate(x, row, col, chunks, n_pad)
    deg = _tc_degsum(pdeg)
    return _tc_combine(x, psum, deg, W.T, b[None, :])


# speedup vs baseline: 3.0046x; 3.0046x over previous
import dataclasses
import functools

import jax
import jax.numpy as jnp
from jax.experimental import pallas as pl
from jax.experimental.pallas import tpu as pltpu
from jax.experimental.pallas import tpu_sc as plsc

NC = 2
NS = 16
LANES = 16
NW = NC * NS


def _sc_aggregate(x, row, col):
    n, d = x.shape
    e = row.shape[0]
    epw = e // NW
    B = 80
    chunks = epw // B
    WB = 80
    nwb = n // WB
    wb_per = (nwb + NS - 1) // NS

    mesh = plsc.VectorSubcoreMesh(
        core_axis_name="c", subcore_axis_name="s",
        num_cores=NC, num_subcores=NS)

    cp = pltpu.CompilerParams()
    if "needs_layout_passes" in pltpu.CompilerParams.__dataclass_fields__:
        cp = dataclasses.replace(cp, needs_layout_passes=False)

    z_feat = jnp.zeros((WB, d), jnp.float32)

    @functools.partial(
        pl.kernel,
        out_type=(jax.ShapeDtypeStruct((NC * n, d), jnp.float32),
                  jax.ShapeDtypeStruct((NW, n), jnp.float32)),
        mesh=mesh,
        compiler_params=cp,
        scratch_types=[
            [pltpu.VMEM((B,), jnp.int32)] * 3,
            [pltpu.VMEM((B,), jnp.int32)] * 3,
            [pltpu.VMEM((B, d), jnp.float32)] * 3,
            pltpu.VMEM((n,), jnp.float32),
            pltpu.VMEM_SHARED((n, d), jnp.float32),
            [pltpu.SemaphoreType.DMA] * 3,
            [pltpu.SemaphoreType.DMA] * 3,
            [pltpu.SemaphoreType.DMA] * 3,
        ],
    )
    def agg_kernel(x_hbm, row_hbm, col_hbm, zf_hbm,
                   sum_hbm, deg_hbm, rb, cb, gb, dloc, acc,
                   sem_r, sem_c, sem_g):
        cid = jax.lax.axis_index("c")
        sid = jax.lax.axis_index("s")
        wid = cid * NS + sid

        @pl.loop(0, wb_per)
        def _(k):
            blk = sid + k * NS

            @pl.when(blk < nwb)
            def _():
                pltpu.sync_copy(zf_hbm, acc.at[pl.ds(blk * WB, WB)])

        @pl.loop(0, n, step=LANES)
        def _(j):
            dloc[pl.ds(j, LANES)] = jnp.zeros((LANES,), jnp.float32)

        plsc.subcore_barrier()

        base = wid * epw
        ones_v = jnp.ones((LANES,), jnp.float32)

        def row_start(k, s):
            pltpu.async_copy(row_hbm.at[pl.ds(base + k * B, B)], rb[s],
                             sem_r[s])

        def col_start(k, s):
            pltpu.async_copy(col_hbm.at[pl.ds(base + k * B, B)], cb[s],
                             sem_c[s])

        def row_wait(s):
            pltpu.make_async_copy(row_hbm.at[pl.ds(base, B)], rb[s],
                                  sem_r[s]).wait()

        def col_wait(s):
            pltpu.make_async_copy(col_hbm.at[pl.ds(base, B)], cb[s],
                                  sem_c[s]).wait()

        def gather_start(s):
            pltpu.async_copy(x_hbm.at[rb[s]], gb[s], sem_g[s])

        def gather_wait(s):
            pltpu.make_async_copy(x_hbm.at[rb[s]], gb[s], sem_g[s]).wait()

        def scatter_deg(s):
            col_wait(s)
            pltpu.sync_copy(gb[s], acc.at[cb[s]], add=True)

            @pl.loop(0, B, step=LANES)
            def _(j):
                plsc.addupdate_scatter(dloc, [cb[s][pl.ds(j, LANES)]], ones_v)

        for s in range(3):
            row_start(s, s)
            col_start(s, s)
        for s in range(3):
            row_wait(s)
            gather_start(s)

        @pl.loop(0, chunks - 2, step=3)
        def _(k):
            for s in range(3):
                c_next = k + s + 3
                gather_wait(s)

                @pl.when(c_next < chunks)
                def _():
                    row_start(c_next, s)

                scatter_deg(s)

                @pl.when(c_next < chunks)
                def _():
                    col_start(c_next, s)
                    row_wait(s)
                    gather_start(s)

        for s in range(2):
            gather_wait(s)
            scatter_deg(s)

        plsc.subcore_barrier()

        @pl.loop(0, wb_per)
        def _(k):
            blk = sid + k * NS

            @pl.when(blk < nwb)
            def _():
                pltpu.sync_copy(acc.at[pl.ds(blk * WB, WB)],
                                sum_hbm.at[pl.ds(cid * n + blk * WB, WB)])

        pltpu.sync_copy(dloc, deg_hbm.at[wid])

    return agg_kernel(x, row, col, z_feat)


def _tc_degsum(pdeg):
    n = pdeg.shape[1]
    ones_nw = jnp.ones((NW, 1), jnp.float32)

    def body(dg_ref, on_ref, o_ref):
        deg = jax.lax.dot_general(
            dg_ref[...], on_ref[...], (((0,), (0,)), ((), ())),
            preferred_element_type=jnp.float32)
        o_ref[...] = jnp.maximum(deg, 1.0)

    return pl.pallas_call(
        body,
        out_shape=jax.ShapeDtypeStruct((n, 1), jnp.float32),
    )(pdeg, ones_nw)


def _tc_combine(x, psum, deg, wt, b2):
    n, d = x.shape
    dout = wt.shape[1]
    bm = 1000
    grid = n // bm

    def body(x_ref, p0_ref, p1_ref, dg_ref, wt_ref, b_ref, o_ref):
        aggr = (p0_ref[...] + p1_ref[...]) / dg_ref[...]
        cat = jnp.concatenate([x_ref[...], aggr], axis=1)
        o_ref[...] = jnp.dot(cat, wt_ref[...],
                             preferred_element_type=jnp.float32) + b_ref[...]

    return pl.pallas_call(
        body,
        grid=(grid,),
        in_specs=[
            pl.BlockSpec((bm, d), lambda i: (i, 0)),
            pl.BlockSpec((bm, d), lambda i: (i, 0)),
            pl.BlockSpec((bm, d), lambda i, g=grid: (i + g, 0)),
            pl.BlockSpec((bm, 1), lambda i: (i, 0)),
            pl.BlockSpec((2 * d, dout), lambda i: (0, 0)),
            pl.BlockSpec((1, dout), lambda i: (0, 0)),
        ],
        out_specs=pl.BlockSpec((bm, dout), lambda i: (i, 0)),
        out_shape=jax.ShapeDtypeStruct((n, dout), jnp.float32),
    )(x, psum, psum, deg, wt, b2)


def kernel(x, edge_index, W, b):
    row = edge_index[0].astype(jnp.int32)
    col = edge_index[1].astype(jnp.int32)
    psum, pdeg = _sc_aggregate(x, row, col)
    deg = _tc_degsum(pdeg)
    return _tc_combine(x, psum, deg, W.T, b[None, :])
